# baseline (speedup 1.0000x reference)
import functools

import jax
import jax.numpy as jnp
from jax.experimental import pallas as pl
from jax.experimental.pallas import tpu as pltpu

_THRESHOLD = 1.5


def _nn_body(q_ref, k_ref, idx_ref, min_ref, *, n_valid, bk, nb):
    i = pl.program_id(0)
    q = q_ref[...]
    k = k_ref[...]
    m = jax.lax.dot_general(
        q, k, (((1,), (1,)), ((), ())),
        preferred_element_type=jnp.float32,
        precision=jax.lax.Precision.DEFAULT,
    )
    q_sq = jnp.sum(q * q, axis=1, keepdims=True)
    k_sq = jnp.sum(k * k, axis=1)[None, :]
    d = (q_sq + k_sq) - 2.0 * m

    lidx = jax.lax.broadcasted_iota(jnp.int32, d.shape, 1)
    limit = jnp.where(i == nb - 1, n_valid - i * bk, bk)
    d = jnp.where(lidx < limit, d, jnp.inf)

    bmin = jnp.min(d, axis=1, keepdims=True)
    cand = jnp.where(d == bmin, lidx, bk)
    barg = jnp.min(cand, axis=1, keepdims=True) + i * bk

    @pl.when(i == 0)
    def _init():
        min_ref[...] = bmin
        idx_ref[...] = barg

    @pl.when(i > 0)
    def _update():
        prev = min_ref[...]
        take = bmin < prev
        min_ref[...] = jnp.where(take, bmin, prev)
        idx_ref[...] = jnp.where(take, barg, idx_ref[...])

    @pl.when(i == nb - 1)
    def _final():
        idx_ref[...] = jnp.where(min_ref[...] > _THRESHOLD, -1, idx_ref[...])


def kernel(source_embs, embeddings):
    q, d_dim = source_embs.shape
    n_k, _ = embeddings.shape
    bk = 2048
    nb = (n_k + bk - 1) // bk
    pad = nb * bk - n_k
    if pad:
        embeddings = jnp.pad(embeddings, ((0, pad), (0, 0)))

    body = functools.partial(_nn_body, n_valid=n_k, bk=bk, nb=nb)
    idx2, min2 = pl.pallas_call(
        body,
        grid=(nb,),
        in_specs=[
            pl.BlockSpec((q, d_dim), lambda i: (0, 0)),
            pl.BlockSpec((bk, d_dim), lambda i: (i, 0)),
        ],
        out_specs=[
            pl.BlockSpec((q, 1), lambda i: (0, 0)),
            pl.BlockSpec((q, 1), lambda i: (0, 0)),
        ],
        out_shape=[
            jax.ShapeDtypeStruct((q, 1), jnp.int32),
            jax.ShapeDtypeStruct((q, 1), jnp.float32),
        ],
        compiler_params=pltpu.CompilerParams(
            dimension_semantics=("arbitrary",),
        ),
    )(source_embs, embeddings)
    return (idx2.reshape(q), min2.reshape(q))
